# baseline (speedup 1.0000x reference)
import functools

import jax
import jax.numpy as jnp
from jax import lax
from jax.experimental import pallas as pl
from jax.experimental.pallas import tpu as pltpu
from jax.experimental.pallas import tpu_sc as plsc

_LANES = 128


def _mm_argmax_body(x_ref, e_ref, dist_ref, ind_ref, *, bp, bn):
    c = e_ref.shape[0]
    n_pan = c // bp
    ch_per_pan = bp // _LANES
    lane = lax.broadcasted_iota(jnp.int32, (bn, _LANES), 1)

    run_max = jnp.full((bn, _LANES), -jnp.inf, jnp.float32)
    run_idx = jnp.zeros((bn, _LANES), jnp.int32)
    for p in range(n_pan):
        d = lax.dot_general(
            x_ref[...], e_ref[pl.ds(p * bp, bp), :],
            dimension_numbers=(((1,), (1,)), ((), ())),
            preferred_element_type=jnp.float32,
        )
        dist_ref[:, pl.ds(p * bp, bp)] = d
        for q in range(ch_per_pan):
            blk = d[:, q * _LANES:(q + 1) * _LANES]
            upd = blk > run_max
            run_max = jnp.maximum(run_max, blk)
            run_idx = jnp.where(upd, lane + (p * bp + q * _LANES), run_idx)

    m = jnp.max(run_max, axis=1)
    loc = jnp.min(jnp.where(run_max == m[:, None], run_idx, jnp.int32(2**30)), axis=1)
    ind_ref[...] = loc


def _matmul_argmax(xf, emb, bn=128, bp=2048):
    n, d = xf.shape
    c = emb.shape[0]
    grid = (n // bn,)
    return pl.pallas_call(
        functools.partial(_mm_argmax_body, bp=bp, bn=bn),
        grid=grid,
        in_specs=[
            pl.BlockSpec((bn, d), lambda i: (i, 0)),
            pl.BlockSpec((c, d), lambda i: (0, 0)),
        ],
        out_specs=[
            pl.BlockSpec((bn, c), lambda i: (i, 0)),
            pl.BlockSpec((bn,), lambda i: (i,)),
        ],
        out_shape=[
            jax.ShapeDtypeStruct((n, c), jnp.float32),
            jax.ShapeDtypeStruct((n,), jnp.int32),
        ],
    )(xf, emb)


def _gather_rows(emb, idx):
    c, d = emb.shape
    b = idx.shape[0]
    info = plsc.get_sparse_core_info()
    nw = info.num_cores * info.num_subcores
    b_per_w = b // nw
    ch = 96
    n_ch = b_per_w // ch
    mesh = plsc.VectorSubcoreMesh(core_axis_name="c", subcore_axis_name="s")

    @functools.partial(
        pl.kernel,
        mesh=mesh,
        out_type=jax.ShapeDtypeStruct((b, d), jnp.float32),
        scratch_types=[
            pltpu.VMEM((n_ch, ch), jnp.int32),
            pltpu.VMEM((b_per_w, d), jnp.float32),
            pltpu.SemaphoreType.DMA,
        ],
    )
    def k(emb_hbm, idx_hbm, out_hbm, idx_v, rows_v, sem):
        wid = lax.axis_index("s") * info.num_cores + lax.axis_index("c")
        for j in range(n_ch):
            pltpu.sync_copy(
                idx_hbm.at[pl.ds(wid * b_per_w + j * ch, ch)], idx_v.at[j]
            )
        copies = [
            pltpu.async_copy(
                emb_hbm.at[idx_v.at[j]], rows_v.at[pl.ds(j * ch, ch)], sem
            )
            for j in range(n_ch)
        ]
        for cp in copies:
            cp.wait()
        pltpu.sync_copy(rows_v, out_hbm.at[pl.ds(wid * b_per_w, b_per_w)])

    return k(emb, idx)


def kernel(x, embeddings):
    x = x.astype(jnp.float32)
    bsz, n, d = x.shape
    emb = embeddings.reshape(embeddings.shape[-2], d)
    xf = x.reshape(bsz * n, d)
    dist, ind = _matmul_argmax(xf, emb)
    quant = _gather_rows(emb, ind)
    quantize = quant.reshape(bsz, n, d)
    embed_ind = ind.reshape(bsz, n)
    dist_unpacked = dist.reshape(1, bsz, n, emb.shape[0])
    return (quantize, embed_ind, dist_unpacked)


# speedup vs baseline: 1.4436x; 1.4436x over previous
import functools

import jax
import jax.numpy as jnp
from jax import lax
from jax.experimental import pallas as pl
from jax.experimental.pallas import tpu as pltpu
from jax.experimental.pallas import tpu_sc as plsc

_LANES = 128


def _mm_argmax_body(x_ref, e_ref, dist_ref, ind_ref, *, bp, bn):
    c = e_ref.shape[0]
    n_pan = c // bp
    ch_per_pan = bp // _LANES
    lane = lax.broadcasted_iota(jnp.int32, (bn, _LANES), 1)

    run_max = jnp.full((bn, _LANES), -jnp.inf, jnp.float32)
    run_idx = jnp.zeros((bn, _LANES), jnp.int32)
    for p in range(n_pan):
        d = lax.dot_general(
            x_ref[...], e_ref[pl.ds(p * bp, bp), :],
            dimension_numbers=(((1,), (1,)), ((), ())),
            preferred_element_type=jnp.float32,
        )
        dist_ref[:, pl.ds(p * bp, bp)] = d
        for q in range(ch_per_pan):
            blk = d[:, q * _LANES:(q + 1) * _LANES]
            upd = blk > run_max
            run_max = jnp.maximum(run_max, blk)
            run_idx = jnp.where(upd, lane + (p * bp + q * _LANES), run_idx)

    m = jnp.max(run_max, axis=1)
    loc = jnp.min(jnp.where(run_max == m[:, None], run_idx, jnp.int32(2**30)), axis=1)
    ind_ref[...] = loc


def _matmul_argmax(xf, emb, bn=256, bp=4096):
    n, d = xf.shape
    c = emb.shape[0]
    grid = (n // bn,)
    return pl.pallas_call(
        functools.partial(_mm_argmax_body, bp=bp, bn=bn),
        grid=grid,
        in_specs=[
            pl.BlockSpec((bn, d), lambda i: (i, 0)),
            pl.BlockSpec((c, d), lambda i: (0, 0)),
        ],
        out_specs=[
            pl.BlockSpec((bn, c), lambda i: (i, 0)),
            pl.BlockSpec((bn,), lambda i: (i,)),
        ],
        out_shape=[
            jax.ShapeDtypeStruct((n, c), jnp.float32),
            jax.ShapeDtypeStruct((n,), jnp.int32),
        ],
    )(xf, emb)


def _gather_rows(emb, idx):
    c, d = emb.shape
    b = idx.shape[0]
    info = plsc.get_sparse_core_info()
    nw = info.num_cores * info.num_subcores
    b_per_w = b // nw
    ch = 96
    n_ch = b_per_w // ch
    mesh = plsc.VectorSubcoreMesh(core_axis_name="c", subcore_axis_name="s")

    @functools.partial(
        pl.kernel,
        mesh=mesh,
        out_type=jax.ShapeDtypeStruct((b, d), jnp.float32),
        scratch_types=[
            pltpu.VMEM((n_ch, ch), jnp.int32),
            pltpu.VMEM((b_per_w, d), jnp.float32),
            pltpu.SemaphoreType.DMA,
        ],
    )
    def k(emb_hbm, idx_hbm, out_hbm, idx_v, rows_v, sem):
        wid = lax.axis_index("s") * info.num_cores + lax.axis_index("c")
        for j in range(n_ch):
            pltpu.sync_copy(
                idx_hbm.at[pl.ds(wid * b_per_w + j * ch, ch)], idx_v.at[j]
            )
        copies = [
            pltpu.async_copy(
                emb_hbm.at[idx_v.at[j]], rows_v.at[pl.ds(j * ch, ch)], sem
            )
            for j in range(n_ch)
        ]
        for cp in copies:
            cp.wait()
        pltpu.sync_copy(rows_v, out_hbm.at[pl.ds(wid * b_per_w, b_per_w)])

    return k(emb, idx)


def kernel(x, embeddings):
    x = x.astype(jnp.float32)
    bsz, n, d = x.shape
    emb = embeddings.reshape(embeddings.shape[-2], d)
    xf = x.reshape(bsz * n, d)
    dist, ind = _matmul_argmax(xf, emb)
    quant = _gather_rows(emb, ind)
    quantize = quant.reshape(bsz, n, d)
    embed_ind = ind.reshape(bsz, n)
    dist_unpacked = dist.reshape(1, bsz, n, emb.shape[0])
    return (quantize, embed_ind, dist_unpacked)
